# Initial kernel scaffold; baseline (speedup 1.0000x reference)
import jax
import jax.numpy as jnp
from jax.experimental import pallas as pl


def kernel(word_seq, batch, edge_index, y, emb_table, W_ih_f, W_hh_f, b_ih_f, b_hh_f, W_ih_b, W_hh_b, b_ih_b, b_hh_b, Wl1, bl1, Wr1, Wl2, bl2, Wr2, Wl3, bl3, Wr3, lin_W, lin_b):
    raise NotImplementedError("write your pallas kernel here")


# speedup vs baseline: 1.1730x; 1.1730x over previous
import functools

import jax
import jax.numpy as jnp
from jax.experimental import pallas as pl
from jax.experimental.pallas import tpu as pltpu

B = 64
T = 384
N = 10000
N_PAD = 10240
PADW = 100000
HID = 128
E = 320000


def _counts_starts_body(batch_ref, counts_ref, starts_ref):
    b2 = batch_ref[...]
    g = jax.lax.broadcasted_iota(jnp.int32, (B, 80, 128), 0)
    eq = (b2[None, :, :] == g).astype(jnp.float32)
    cnt = jnp.sum(eq, axis=(1, 2)).reshape(B, 1)
    r = jax.lax.broadcasted_iota(jnp.int32, (B, B), 0)
    c = jax.lax.broadcasted_iota(jnp.int32, (B, B), 1)
    tri = (c < r).astype(jnp.float32)
    counts_ref[...] = cnt
    starts_ref[...] = jnp.dot(tri, cnt, preferred_element_type=jnp.float32)


def _counts_starts(batch_pad):
    return pl.pallas_call(
        _counts_starts_body,
        out_shape=(
            jax.ShapeDtypeStruct((B, 1), jnp.float32),
            jax.ShapeDtypeStruct((B, 1), jnp.float32),
        ),
    )(batch_pad.reshape(80, 128))


def _bilstm_body(xf_ref, xb_ref, A_ref, bias_ref, outf_ref, outb_ref,
                 hf, cf, hb, cb):
    t = pl.program_id(0)

    @pl.when(t == 0)
    def _():
        hf[...] = jnp.zeros_like(hf)
        cf[...] = jnp.zeros_like(cf)
        hb[...] = jnp.zeros_like(hb)
        cb[...] = jnp.zeros_like(cb)

    xf = xf_ref[0]
    xb = xb_ref[0]
    cat = jnp.concatenate([xf, hf[...], xb, hb[...]], axis=1)
    g = jnp.dot(cat, A_ref[...], preferred_element_type=jnp.float32)
    g = g + bias_ref[0][None, :]
    gf, gb = g[:, :4 * HID], g[:, 4 * HID:]

    def cell(gg, c_prev):
        i = jax.nn.sigmoid(gg[:, 0:HID])
        f = jax.nn.sigmoid(gg[:, HID:2 * HID])
        u = jnp.tanh(gg[:, 2 * HID:3 * HID])
        o = jax.nn.sigmoid(gg[:, 3 * HID:4 * HID])
        c_new = f * c_prev + i * u
        return jax.nn.sigmoid(gg[:, 3 * HID:4 * HID]) * jnp.tanh(c_new), c_new, o

    hf_new, cf_new, _ = cell(gf, cf[...])
    hb_new, cb_new, _ = cell(gb, cb[...])
    hf[...] = hf_new
    cf[...] = cf_new
    hb[...] = hb_new
    cb[...] = cb_new
    outf_ref[0] = hf_new
    outb_ref[0] = hb_new


def _bilstm(feats_t, A, bias):
    return pl.pallas_call(
        _bilstm_body,
        grid=(T,),
        in_specs=[
            pl.BlockSpec((1, B, HID), lambda t: (t, 0, 0)),
            pl.BlockSpec((1, B, HID), lambda t: (T - 1 - t, 0, 0)),
            pl.BlockSpec((4 * HID, 8 * HID), lambda t: (0, 0)),
            pl.BlockSpec((1, 8 * HID), lambda t: (0, 0)),
        ],
        out_specs=[
            pl.BlockSpec((1, B, HID), lambda t: (t, 0, 0)),
            pl.BlockSpec((1, B, HID), lambda t: (T - 1 - t, 0, 0)),
        ],
        out_shape=[
            jax.ShapeDtypeStruct((T, B, HID), jnp.float32),
            jax.ShapeDtypeStruct((T, B, HID), jnp.float32),
        ],
        scratch_shapes=[pltpu.VMEM((B, HID), jnp.float32)] * 4,
    )(feats_t, feats_t, A, bias)


def _sage_mm_body(s_ref, x_ref, cnt_ref, W_ref, b_ref, out_ref):
    invc = 1.0 / jnp.maximum(cnt_ref[...], 1.0)
    a = s_ref[...] * invc
    cat = jnp.concatenate([a, x_ref[...]], axis=1)
    o = jnp.dot(cat, W_ref[...], preferred_element_type=jnp.float32)
    o = o + b_ref[0][None, :]
    nrm = jnp.sqrt(jnp.sum(o * o, axis=1, keepdims=True))
    o = o / jnp.maximum(nrm, 1e-12)
    out_ref[...] = jnp.maximum(o, 0.0)


def _sage_mm(s, x, cnt, Wcat, b):
    d = x.shape[1]
    dout = Wcat.shape[1]
    grid = N_PAD // 128
    return pl.pallas_call(
        _sage_mm_body,
        grid=(grid,),
        in_specs=[
            pl.BlockSpec((128, d), lambda i: (i, 0)),
            pl.BlockSpec((128, d), lambda i: (i, 0)),
            pl.BlockSpec((128, 1), lambda i: (i, 0)),
            pl.BlockSpec((2 * d, dout), lambda i: (0, 0)),
            pl.BlockSpec((1, dout), lambda i: (0, 0)),
        ],
        out_specs=pl.BlockSpec((128, dout), lambda i: (i, 0)),
        out_shape=jax.ShapeDtypeStruct((N_PAD, dout), jnp.float32),
    )(s, x, cnt, Wcat, b)


def _pool_body(x_ref, batch_ref, gcnt_ref, Wt_ref, b_ref, out_ref, acc):
    i = pl.program_id(0)

    @pl.when(i == 0)
    def _():
        acc[...] = jnp.zeros_like(acc)

    brow = batch_ref[0, 0, :]
    P = (jax.lax.broadcasted_iota(jnp.int32, (B, 128), 0)
         == brow[None, :]).astype(jnp.float32)
    acc[...] += jnp.dot(P, x_ref[...], preferred_element_type=jnp.float32)

    @pl.when(i == N_PAD // 128 - 1)
    def _():
        pooled = acc[...] / jnp.maximum(gcnt_ref[...], 1.0)
        out_ref[...] = (jnp.dot(pooled, Wt_ref[...],
                                preferred_element_type=jnp.float32)
                        + b_ref[0][None, :])


def _pool_linear(x, batch3, gcnt, Wt, b):
    nc = Wt.shape[1]
    grid = N_PAD // 128
    return pl.pallas_call(
        _pool_body,
        grid=(grid,),
        in_specs=[
            pl.BlockSpec((128, 512), lambda i: (i, 0)),
            pl.BlockSpec((1, 1, 128), lambda i: (i, 0, 0)),
            pl.BlockSpec((B, 1), lambda i: (0, 0)),
            pl.BlockSpec((512, nc), lambda i: (0, 0)),
            pl.BlockSpec((1, nc), lambda i: (0, 0)),
        ],
        out_specs=pl.BlockSpec((B, nc), lambda i: (0, 0)),
        out_shape=jax.ShapeDtypeStruct((B, nc), jnp.float32),
        scratch_shapes=[pltpu.VMEM((B, 512), jnp.float32)],
    )(x, batch3, gcnt, Wt, b)


def kernel(word_seq, batch, edge_index, y, emb_table,
           W_ih_f, W_hh_f, b_ih_f, b_hh_f,
           W_ih_b, W_hh_b, b_ih_b, b_hh_b,
           Wl1, bl1, Wr1, Wl2, bl2, Wr2, Wl3, bl3, Wr3, lin_W, lin_b):
    batch_pad = jnp.pad(batch, (0, N_PAD - N), constant_values=B)
    counts_f, starts_f = _counts_starts(batch_pad)
    counts = counts_f[:, 0].astype(jnp.int32)
    starts = starts_f[:, 0].astype(jnp.int32)

    tt = jnp.arange(T, dtype=jnp.int32)[None, :]
    idx = starts[:, None] + tt
    valid = tt < counts[:, None]
    words = jnp.where(valid, word_seq[jnp.clip(idx, 0, N - 1)], PADW)
    feats = emb_table[words]
    feats_t = jnp.transpose(feats, (1, 0, 2))

    Af = jnp.concatenate([W_ih_f.T, W_hh_f.T], axis=0)
    Ab = jnp.concatenate([W_ih_b.T, W_hh_b.T], axis=0)
    A = jnp.zeros((4 * HID, 8 * HID), jnp.float32)
    A = A.at[:2 * HID, :4 * HID].set(Af).at[2 * HID:, 4 * HID:].set(Ab)
    bias = jnp.concatenate([b_ih_f + b_hh_f, b_ih_b + b_hh_b])[None, :]

    hf, hb = _bilstm(feats_t, A, bias)

    pos = jnp.arange(N, dtype=jnp.int32) - starts[batch]
    pos = jnp.clip(pos, 0, T - 1)
    r = pos * B + batch
    x = jnp.concatenate(
        [hf.reshape(T * B, HID)[r], hb.reshape(T * B, HID)[r]], axis=1)

    src, dst = edge_index[0], edge_index[1]
    deg = jax.ops.segment_sum(jnp.ones((E,), jnp.float32), dst,
                              num_segments=N)
    deg_p = jnp.pad(deg, (0, N_PAD - N))[:, None]

    def sage(x10k, Wl, bl, Wr):
        s = jax.ops.segment_sum(x10k[src], dst, num_segments=N)
        d = x10k.shape[1]
        s_p = jnp.pad(s, ((0, N_PAD - N), (0, 0)))
        x_p = jnp.pad(x10k, ((0, N_PAD - N), (0, 0)))
        Wcat = jnp.concatenate([Wl.T, Wr.T], axis=0)
        out = _sage_mm(s_p, x_p, deg_p, Wcat, bl[None, :])
        return out[:N]

    x = sage(x, Wl1, bl1, Wr1)
    x = sage(x, Wl2, bl2, Wr2)
    x = sage(x, Wl3, bl3, Wr3)

    x_p = jnp.pad(x, ((0, N_PAD - N), (0, 0)))
    batch3 = batch_pad.reshape(80, 1, 128)
    return _pool_linear(x_p, batch3, counts_f, lin_W.T, lin_b[None, :])
